# baseline (speedup 1.0000x reference)
import functools

import jax
import jax.numpy as jnp
from jax import lax
from jax.experimental import pallas as pl
from jax.experimental.pallas import tpu as pltpu
from jax.experimental.pallas import tpu_sc as plsc

NUM_CORES = 2
NUM_SUBCORES = 16
NUM_WORKERS = NUM_CORES * NUM_SUBCORES

B = 4096
H = 200
D = 64
BW = B // NUM_WORKERS
HT, HS = 25, 8
DT, DS = 8, 8
NBUF = 2


def _body(feat_hbm, idx_hbm, table_hbm, out_hbm, idx_v, feat_v, rows_v, out_v,
          *sems):
    lsem = sems[0:NBUF]
    gsem = sems[NBUF:2 * NBUF]
    ssem = sems[2 * NBUF:3 * NBUF]

    c = lax.axis_index("c")
    s = lax.axis_index("s")
    wid = s * NUM_CORES + c

    def load_issue(h, b):
        pltpu.async_copy(feat_hbm.at[h, :, wid], feat_v.at[b], lsem[b])

    def load_wait(h, b):
        pltpu.make_async_copy(feat_hbm.at[h, :, wid], feat_v.at[b],
                              lsem[b]).wait()

    def gather_issue(ht, hs, b):
        pltpu.async_copy(table_hbm.at[idx_v.at[ht, hs]], rows_v.at[b], gsem[b])

    def gather_wait(ht, hs, b):
        pltpu.make_async_copy(table_hbm.at[idx_v.at[ht, hs]], rows_v.at[b],
                              gsem[b]).wait()

    def store_issue(h, b):
        pltpu.async_copy(out_v.at[b], out_hbm.at[h, :, wid], ssem[b])

    def store_wait(h, b):
        pltpu.make_async_copy(out_v.at[b], out_hbm.at[h, :, wid],
                              ssem[b]).wait()

    pltpu.sync_copy(idx_hbm.at[:, wid], idx_v)

    lane = lax.iota(jnp.int32, 16)

    def compute(b):
        bvec = jnp.full((16,), b, jnp.int32)

        def dstep(dt, carry):
            for ds in range(DS):
                dvec = jnp.full((16,), 0, jnp.int32) + (dt * DS + ds)
                for g in range(BW // 16):
                    fv = feat_v[b, dt, ds, pl.ds(g * 16, 16)]
                    rv = plsc.load_gather(rows_v, [bvec, g * 16 + lane, dvec])
                    out_v[b, dt, ds, pl.ds(g * 16, 16)] = fv + rv
            return carry

        lax.fori_loop(0, DT, dstep, 0)

    load_issue(0, 0)
    gather_issue(0, 0, 0)

    def group(ht, carry):
        for hs in range(HS):
            h = ht * HS + hs
            b = hs % NBUF
            bn = (hs + 1) % NBUF

            @pl.when(h + 1 < H)
            def _():
                load_issue(h + 1, bn)
                gather_issue(ht + (hs + 1) // HS, (hs + 1) % HS, bn)

            load_wait(h, b)
            gather_wait(ht, hs, b)

            @pl.when(h >= NBUF)
            def _():
                store_wait(h - NBUF, b)

            compute(b)
            store_issue(h, b)
        return carry

    lax.fori_loop(0, HT, group, 0)

    for j in range(H - NBUF, H):
        store_wait(j, j % NBUF)


@jax.jit
def _run(feat5, idx4, table):
    mesh = plsc.VectorSubcoreMesh(core_axis_name="c", subcore_axis_name="s")
    kern = pl.kernel(
        _body,
        out_type=jax.ShapeDtypeStruct((H, DT, NUM_WORKERS, DS, BW),
                                      jnp.float32),
        mesh=mesh,
        scratch_types=[
            pltpu.VMEM((HT, HS, BW), jnp.int32),
            pltpu.VMEM((NBUF, DT, DS, BW), jnp.float32),
            pltpu.VMEM((NBUF, BW, D), jnp.float32),
            pltpu.VMEM((NBUF, DT, DS, BW), jnp.float32),
        ] + [pltpu.SemaphoreType.DMA] * (3 * NBUF),
        compiler_params=pltpu.CompilerParams(
            use_tc_tiling_on_sc=False, needs_layout_passes=False),
    )
    return kern(feat5, idx4, table)


def kernel(features, indices, table):
    feat5 = features.reshape(NUM_WORKERS, BW, H, DT, DS).transpose(
        2, 3, 0, 4, 1)
    idx4 = indices.astype(jnp.int32).reshape(
        NUM_WORKERS, BW, HT, HS).transpose(2, 0, 3, 1)
    out5 = _run(feat5, idx4, table)
    return out5.transpose(2, 4, 0, 1, 3).reshape(B, H, D)


# speedup vs baseline: 1.5440x; 1.5440x over previous
import functools

import jax
import jax.numpy as jnp
from jax import lax
from jax.experimental import pallas as pl
from jax.experimental.pallas import tpu as pltpu
from jax.experimental.pallas import tpu_sc as plsc

NUM_CORES = 2
NUM_SUBCORES = 16
NUM_WORKERS = NUM_CORES * NUM_SUBCORES

B = 4096
H = 200
D = 64
V = 1000000
BW = B // NUM_WORKERS
HT, HS = 25, 8
DT, DS = 8, 8
NBUF = 4
CB = 2048


def _sc_body(idx_hbm, table_hbm, g_hbm, idx_v, rows_v, *sems):
    gsem = sems[0:NBUF]
    ssem = sems[NBUF:2 * NBUF]

    c = lax.axis_index("c")
    s = lax.axis_index("s")
    wid = s * NUM_CORES + c

    def gather_issue(ht, hs, b):
        pltpu.async_copy(table_hbm.at[idx_v.at[ht, hs]], rows_v.at[b], gsem[b])

    def gather_wait(ht, hs, b):
        pltpu.make_async_copy(table_hbm.at[idx_v.at[ht, hs]], rows_v.at[b],
                              gsem[b]).wait()

    def store_issue(h, b):
        pltpu.async_copy(rows_v.at[b],
                         g_hbm.at[h * NUM_WORKERS + wid, :, pl.ds(0, D)],
                         ssem[b])

    def store_wait(h, b):
        pltpu.make_async_copy(rows_v.at[b],
                              g_hbm.at[h * NUM_WORKERS + wid, :, pl.ds(0, D)],
                              ssem[b]).wait()

    pltpu.sync_copy(idx_hbm.at[:, wid], idx_v)

    gather_issue(0, 0, 0)

    def group(ht, carry):
        for hs in range(HS):
            h = ht * HS + hs
            b = hs % NBUF
            bn = (hs + 1) % NBUF

            @pl.when(h + 1 < H)
            def _():
                @pl.when(h >= NBUF - 1)
                def _():
                    store_wait(h + 1 - NBUF, bn)
                gather_issue(ht + (hs + 1) // HS, (hs + 1) % HS, bn)

            gather_wait(ht, hs, b)
            store_issue(h, b)
        return carry

    lax.fori_loop(0, HT, group, 0)

    for j in range(H - NBUF, H):
        store_wait(j, j % NBUF)


@jax.jit
def _sc_gather(idx4, table):
    mesh = plsc.VectorSubcoreMesh(core_axis_name="c", subcore_axis_name="s")
    kern = pl.kernel(
        _sc_body,
        out_type=jax.ShapeDtypeStruct((H * NUM_WORKERS, BW, 2 * D),
                                      jnp.float32),
        mesh=mesh,
        scratch_types=[
            pltpu.VMEM((HT, HS, BW), jnp.int32),
            pltpu.VMEM((NBUF, BW, D), jnp.float32),
        ] + [pltpu.SemaphoreType.DMA] * (2 * NBUF),
        compiler_params=pltpu.CompilerParams(
            use_tc_tiling_on_sc=False, needs_layout_passes=False),
    )
    return kern(idx4, table)


def _tc_prep_body(tt_ref, out_ref):
    t = jnp.swapaxes(tt_ref[...], 0, 1)
    out_ref[...] = t.reshape(CB // 8, 8, D)


@jax.jit
def _tc_prep(tableT):
    return pl.pallas_call(
        _tc_prep_body,
        grid=(pl.cdiv(V, CB),),
        in_specs=[pl.BlockSpec((D, CB), lambda j: (0, j))],
        out_specs=pl.BlockSpec((CB // 8, 8, D), lambda j: (j, 0, 0)),
        out_shape=jax.ShapeDtypeStruct((V // 8, 8, D), jnp.float32),
    )(tableT)


def _tc_body(feat_ref, g_ref, out_ref):
    g = g_ref[...][:, :, :D]
    gt = jnp.swapaxes(g, 1, 2)
    gt = gt.reshape(NUM_WORKERS, DT, DS, BW)
    gt = jnp.transpose(gt, (1, 0, 2, 3))
    out_ref[0] = feat_ref[0] + gt


@jax.jit
def _tc_add(feat5, g):
    return pl.pallas_call(
        _tc_body,
        grid=(H,),
        in_specs=[
            pl.BlockSpec((1, DT, NUM_WORKERS, DS, BW),
                         lambda i: (i, 0, 0, 0, 0)),
            pl.BlockSpec((NUM_WORKERS, BW, 2 * D), lambda i: (i, 0, 0)),
        ],
        out_specs=pl.BlockSpec((1, DT, NUM_WORKERS, DS, BW),
                               lambda i: (i, 0, 0, 0, 0)),
        out_shape=jax.ShapeDtypeStruct((H, DT, NUM_WORKERS, DS, BW),
                                       jnp.float32),
    )(feat5, g)


def kernel(features, indices, table):
    feat5 = features.reshape(NUM_WORKERS, BW, H, DT, DS).transpose(
        2, 3, 0, 4, 1)
    idx4 = indices.astype(jnp.int32).reshape(
        NUM_WORKERS, BW, HT, HS).transpose(2, 0, 3, 1)
    tableT = jnp.swapaxes(table, 0, 1)
    table_c = _tc_prep(tableT).reshape(V, D)
    g = _sc_gather(idx4, table_c)
    out5 = _tc_add(feat5, g)
    return out5.transpose(2, 4, 0, 1, 3).reshape(B, H, D)


# speedup vs baseline: 1.6812x; 1.0889x over previous
import functools

import jax
import jax.numpy as jnp
from jax import lax
from jax.experimental import pallas as pl
from jax.experimental.pallas import tpu as pltpu
from jax.experimental.pallas import tpu_sc as plsc

NUM_CORES = 2
NUM_SUBCORES = 16
NUM_WORKERS = NUM_CORES * NUM_SUBCORES

B = 4096
H = 200
D = 64
V = 1000000
BW = B // NUM_WORKERS
HT, HS = 25, 8
DT, DS = 8, 8
NBUF = 4


def _sc_body(idx_hbm, table_hbm, g_hbm, idx_v, rows_v, *sems):
    gsem = sems[0:NBUF]
    ssem = sems[NBUF:2 * NBUF]

    c = lax.axis_index("c")
    s = lax.axis_index("s")
    wid = s * NUM_CORES + c

    def gather_issue(ht, hs, b):
        pltpu.async_copy(table_hbm.at[idx_v.at[ht, hs]], rows_v.at[b], gsem[b])

    def gather_wait(ht, hs, b):
        pltpu.make_async_copy(table_hbm.at[idx_v.at[ht, hs]], rows_v.at[b],
                              gsem[b]).wait()

    def store_issue(h, b):
        pltpu.async_copy(rows_v.at[b], g_hbm.at[h * NUM_WORKERS + wid],
                         ssem[b])

    def store_wait(h, b):
        pltpu.make_async_copy(rows_v.at[b], g_hbm.at[h * NUM_WORKERS + wid],
                              ssem[b]).wait()

    pltpu.sync_copy(idx_hbm.at[:, wid], idx_v)

    gather_issue(0, 0, 0)

    def group(ht, carry):
        for hs in range(HS):
            h = ht * HS + hs
            b = hs % NBUF
            bn = (hs + 1) % NBUF

            @pl.when(h + 1 < H)
            def _():
                @pl.when(h >= NBUF - 1)
                def _():
                    store_wait(h + 1 - NBUF, bn)
                gather_issue(ht + (hs + 1) // HS, (hs + 1) % HS, bn)

            gather_wait(ht, hs, b)
            store_issue(h, b)
        return carry

    lax.fori_loop(0, HT, group, 0)

    for j in range(H - NBUF, H):
        store_wait(j, j % NBUF)


@jax.jit
def _sc_gather(idxh4, table2):
    mesh = plsc.VectorSubcoreMesh(core_axis_name="c", subcore_axis_name="s")
    kern = pl.kernel(
        _sc_body,
        out_type=jax.ShapeDtypeStruct((H * NUM_WORKERS, BW, 2 * D),
                                      jnp.float32),
        mesh=mesh,
        scratch_types=[
            pltpu.VMEM((HT, HS, BW), jnp.int32),
            pltpu.VMEM((NBUF, BW, 2 * D), jnp.float32),
        ] + [pltpu.SemaphoreType.DMA] * (2 * NBUF),
        compiler_params=pltpu.CompilerParams(
            use_tc_tiling_on_sc=False, needs_layout_passes=False),
    )
    return kern(idxh4, table2)


def _tc_body(feat_ref, g_ref, par_ref, out_ref):
    g = g_ref[...]
    gt = jnp.swapaxes(g, 1, 2)
    par = par_ref[0]
    parb = jnp.broadcast_to(par[:, None, :], (NUM_WORKERS, D, BW))
    gsel = jnp.where(parb == 1, gt[:, D:, :], gt[:, :D, :])
    gsel = gsel.reshape(NUM_WORKERS, DT, DS, BW)
    gsel = jnp.transpose(gsel, (1, 0, 2, 3))
    out_ref[0] = feat_ref[0] + gsel


@jax.jit
def _tc_add(feat5, g, par4):
    return pl.pallas_call(
        _tc_body,
        grid=(H,),
        in_specs=[
            pl.BlockSpec((1, DT, NUM_WORKERS, DS, BW),
                         lambda i: (i, 0, 0, 0, 0)),
            pl.BlockSpec((NUM_WORKERS, BW, 2 * D), lambda i: (i, 0, 0)),
            pl.BlockSpec((1, NUM_WORKERS, BW), lambda i: (i, 0, 0)),
        ],
        out_specs=pl.BlockSpec((1, DT, NUM_WORKERS, DS, BW),
                               lambda i: (i, 0, 0, 0, 0)),
        out_shape=jax.ShapeDtypeStruct((H, DT, NUM_WORKERS, DS, BW),
                                       jnp.float32),
    )(feat5, g, par4)


def kernel(features, indices, table):
    feat5 = features.reshape(NUM_WORKERS, BW, H, DT, DS).transpose(
        2, 3, 0, 4, 1)
    idx4 = indices.astype(jnp.int32).reshape(
        NUM_WORKERS, BW, HT, HS).transpose(2, 0, 3, 1)
    idxh4 = idx4 >> 1
    par4 = jnp.transpose(idx4 & 1, (0, 2, 1, 3)).reshape(
        H, NUM_WORKERS, BW)
    table2 = table.reshape(V // 2, 2 * D)
    g = _sc_gather(idxh4, table2)
    out5 = _tc_add(feat5, g, par4)
    return out5.transpose(2, 4, 0, 1, 3).reshape(B, H, D)
